# baseline (speedup 1.0000x reference)
import jax
import jax.numpy as jnp
from jax import lax
from jax.experimental import pallas as pl
from jax.experimental.pallas import tpu as pltpu
from jax.experimental.pallas import tpu_sc as plsc

_NUM_CORES = 2
_NUM_SUBCORES = 16
_NUM_WORKERS = _NUM_CORES * _NUM_SUBCORES
_LANES = 16

_N_EDGES = 320000
_N_NODES = 10000
_D = 128
_B = 80
_EPW = _N_EDGES // _NUM_WORKERS
_NBLK = _EPW // _B
_ROWS_PER_TILE = 624


def _sc_body(z_hbm, src_hbm, dst_hbm, out_hbm,
             si0, si1, di0, di1, sr0, sr1, dr0, dr1, ob0, ob1, z_sh,
             tr0, tr1,
             sem_i0, sem_i1, sem_g0, sem_g1, sem_o0, sem_o1):
  sid = lax.axis_index("s")
  wid = lax.axis_index("c") * _NUM_SUBCORES + sid
  base = wid * _EPW

  si = (si0, si1)
  di = (di0, di1)
  sr = (sr0, sr1)
  dr = (dr0, dr1)
  ob = (ob0, ob1)
  sem_i = (sem_i0, sem_i1)
  sem_g = (sem_g0, sem_g1)
  sem_o = (sem_o0, sem_o1)

  r0 = sid * _ROWS_PER_TILE
  pltpu.sync_copy(z_hbm.at[pl.ds(r0, _ROWS_PER_TILE)],
                  z_sh.at[pl.ds(r0, _ROWS_PER_TILE)])
  rem0 = _NUM_SUBCORES * _ROWS_PER_TILE

  @pl.when(sid == 0)
  def _():
    pltpu.sync_copy(z_hbm.at[pl.ds(rem0, _N_NODES - rem0)],
                    z_sh.at[pl.ds(rem0, _N_NODES - rem0)])

  plsc.subcore_barrier()

  def start_idx(blk, p):
    off = base + blk * _B
    pltpu.async_copy(src_hbm.at[pl.ds(off, _B)], si[p], sem_i[p])
    pltpu.async_copy(dst_hbm.at[pl.ds(off, _B)], di[p], sem_i[p])

  def wait_idx(p):
    pltpu.make_async_copy(src_hbm.at[pl.ds(0, _B)], si[p], sem_i[p]).wait()
    pltpu.make_async_copy(dst_hbm.at[pl.ds(0, _B)], di[p], sem_i[p]).wait()

  def start_gather(p):
    pltpu.async_copy(z_sh.at[si[p]], sr[p], sem_g[p])
    pltpu.async_copy(z_sh.at[di[p]], dr[p], sem_g[p])

  def wait_gather(p):
    pltpu.make_async_copy(z_sh.at[si[p]], sr[p], sem_g[p]).wait()
    pltpu.make_async_copy(z_sh.at[di[p]], dr[p], sem_g[p]).wait()

  def start_out(blk, p):
    pltpu.async_copy(ob[p], out_hbm.at[pl.ds(base + blk * _B, _B)], sem_o[p])

  def wait_out(p):
    pltpu.make_async_copy(ob[p], out_hbm.at[pl.ds(0, _B)], sem_o[p]).wait()

  def compute(p, tr0, tr1):
    s_rows = sr[p]
    d_rows = dr[p]
    lane = lax.iota(jnp.int32, _LANES)

    def one_group(e0, tr):
      for j in range(_LANES):
        acc = (s_rows[e0 + j, pl.ds(0, _LANES)]
               * d_rows[e0 + j, pl.ds(0, _LANES)])
        for k in range(1, _D // _LANES):
          acc = acc + (s_rows[e0 + j, pl.ds(k * _LANES, _LANES)]
                       * d_rows[e0 + j, pl.ds(k * _LANES, _LANES)])
        tr[j, pl.ds(0, _LANES)] = acc
      u = [plsc.load_gather(tr, [lane, jnp.full((_LANES,), k, jnp.int32)])
           for k in range(_LANES)]
      while len(u) > 1:
        u = [u[i2] + u[i2 + 1] for i2 in range(0, len(u), 2)]
      ob[p][pl.ds(e0, _LANES)] = 1.0 / (1.0 + jnp.exp(-u[0]))

    def grp_pair(i, carry2):
      one_group(2 * i * _LANES, tr0)
      one_group((2 * i + 1) * _LANES, tr1)
      return carry2

    lax.fori_loop(0, _B // (2 * _LANES), grp_pair, 0)
    one_group(_B - _LANES, tr0)

  start_idx(0, 0)
  start_idx(1, 1)
  wait_idx(0)
  start_gather(0)

  def it_body(i, carry):
    for p in range(2):
      blk = 2 * i + p

      @pl.when(blk < _NBLK)
      def _():
        wait_gather(p)

        @pl.when(blk + 2 < _NBLK)
        def _():
          start_idx(blk + 2, p)

        @pl.when(blk + 1 < _NBLK)
        def _():
          wait_idx(1 - p)
          start_gather(1 - p)

        @pl.when(blk >= 2)
        def _():
          wait_out(p)

        compute(p, tr0, tr1)
        start_out(blk, p)

    return carry

  lax.fori_loop(0, (_NBLK + 1) // 2, it_body, 0)

  wait_out((_NBLK - 2) % 2)
  wait_out((_NBLK - 1) % 2)


@jax.jit
def _decode(z, src, dst):
  mesh = plsc.VectorSubcoreMesh(
      core_axis_name="c", subcore_axis_name="s",
      num_cores=_NUM_CORES, num_subcores=_NUM_SUBCORES)
  return pl.kernel(
      _sc_body,
      out_type=jax.ShapeDtypeStruct((_N_EDGES,), jnp.float32),
      mesh=mesh,
      compiler_params=pltpu.CompilerParams(needs_layout_passes=False),
      scratch_types=[
          pltpu.VMEM((_B,), jnp.int32),
          pltpu.VMEM((_B,), jnp.int32),
          pltpu.VMEM((_B,), jnp.int32),
          pltpu.VMEM((_B,), jnp.int32),
          pltpu.VMEM((_B, _D), jnp.float32),
          pltpu.VMEM((_B, _D), jnp.float32),
          pltpu.VMEM((_B, _D), jnp.float32),
          pltpu.VMEM((_B, _D), jnp.float32),
          pltpu.VMEM((_B,), jnp.float32),
          pltpu.VMEM((_B,), jnp.float32),
          pltpu.VMEM_SHARED((_N_NODES, _D), jnp.float32),
          pltpu.VMEM((_LANES, _LANES + 1), jnp.float32),
          pltpu.VMEM((_LANES, _LANES + 1), jnp.float32),
          pltpu.SemaphoreType.DMA,
          pltpu.SemaphoreType.DMA,
          pltpu.SemaphoreType.DMA,
          pltpu.SemaphoreType.DMA,
          pltpu.SemaphoreType.DMA,
          pltpu.SemaphoreType.DMA,
      ],
  )(z, src, dst)


def kernel(z, edge_index):
  src = edge_index[0]
  dst = edge_index[1]
  return _decode(z, src, dst)


# speedup vs baseline: 1.4361x; 1.4361x over previous
import jax
import jax.numpy as jnp
from jax import lax
from jax.experimental import pallas as pl
from jax.experimental.pallas import tpu as pltpu
from jax.experimental.pallas import tpu_sc as plsc

_NUM_CORES = 2
_NUM_SUBCORES = 16
_NUM_WORKERS = _NUM_CORES * _NUM_SUBCORES
_LANES = 16

_N_EDGES = 320000
_N_NODES = 10000
_D = 128
_B = 80
_EPW = _N_EDGES // _NUM_WORKERS
_NBLK = _EPW // _B
_ROWS_PER_TILE = 624


def _sc_body(z_hbm, src_hbm, dst_hbm, out_hbm,
             si0, si1, di0, di1, sr0, sr1, dr0, dr1, ob0, ob1, z_sh,
             tr0, tr1,
             sem_i0, sem_i1, sem_g0, sem_g1, sem_o0, sem_o1):
  sid = lax.axis_index("s")
  wid = lax.axis_index("c") * _NUM_SUBCORES + sid
  base = wid * _EPW

  si = (si0, si1)
  di = (di0, di1)
  sr = (sr0, sr1)
  dr = (dr0, dr1)
  ob = (ob0, ob1)
  sem_i = (sem_i0, sem_i1)
  sem_g = (sem_g0, sem_g1)
  sem_o = (sem_o0, sem_o1)

  r0 = sid * _ROWS_PER_TILE
  pltpu.sync_copy(z_hbm.at[pl.ds(r0, _ROWS_PER_TILE)],
                  z_sh.at[pl.ds(r0, _ROWS_PER_TILE)])
  rem0 = _NUM_SUBCORES * _ROWS_PER_TILE

  @pl.when(sid == 0)
  def _():
    pltpu.sync_copy(z_hbm.at[pl.ds(rem0, _N_NODES - rem0)],
                    z_sh.at[pl.ds(rem0, _N_NODES - rem0)])

  plsc.subcore_barrier()

  def start_idx(blk, p):
    off = base + blk * _B
    pltpu.async_copy(src_hbm.at[pl.ds(off, _B)], si[p], sem_i[p])
    pltpu.async_copy(dst_hbm.at[pl.ds(off, _B)], di[p], sem_i[p])

  def wait_idx(p):
    pltpu.make_async_copy(src_hbm.at[pl.ds(0, _B)], si[p], sem_i[p]).wait()
    pltpu.make_async_copy(dst_hbm.at[pl.ds(0, _B)], di[p], sem_i[p]).wait()

  def start_gather(p):
    pltpu.async_copy(z_sh.at[si[p]], sr[p], sem_g[p])
    pltpu.async_copy(z_sh.at[di[p]], dr[p], sem_g[p])

  def wait_gather(p):
    pltpu.make_async_copy(z_sh.at[si[p]], sr[p], sem_g[p]).wait()
    pltpu.make_async_copy(z_sh.at[di[p]], dr[p], sem_g[p]).wait()

  def start_out(blk, p):
    pltpu.async_copy(ob[p], out_hbm.at[pl.ds(base + blk * _B, _B)], sem_o[p])

  def wait_out(p):
    pltpu.make_async_copy(ob[p], out_hbm.at[pl.ds(0, _B)], sem_o[p]).wait()

  def compute(p, tr0, tr1):
    s_rows = sr[p]
    d_rows = dr[p]
    lane = lax.iota(jnp.int32, _LANES)

    def edge_body(j, carry2):
      acc = (s_rows[j, pl.ds(0, _LANES)] * d_rows[j, pl.ds(0, _LANES)])
      for k in range(1, _D // _LANES):
        acc = acc + (s_rows[j, pl.ds(k * _LANES, _LANES)]
                     * d_rows[j, pl.ds(k * _LANES, _LANES)])
      tr0[j % _LANES, pl.ds(0, _LANES)] = acc
      return carry2

    def grp_body(g, carry2):
      e0 = g * _LANES
      lax.fori_loop(e0, e0 + _LANES, edge_body, 0)
      u = [plsc.load_gather(tr0, [lane, jnp.full((_LANES,), k, jnp.int32)])
           for k in range(_LANES)]
      while len(u) > 1:
        u = [u[i2] + u[i2 + 1] for i2 in range(0, len(u), 2)]
      ob[p][pl.ds(e0, _LANES)] = 1.0 / (1.0 + jnp.exp(-u[0]))
      return carry2

    lax.fori_loop(0, _B // _LANES, grp_body, 0)

  start_idx(0, 0)
  start_idx(1, 1)
  wait_idx(0)
  start_gather(0)

  def it_body(i, carry):
    for p in range(2):
      blk = 2 * i + p

      @pl.when(blk < _NBLK)
      def _():
        wait_gather(p)

        @pl.when(blk + 2 < _NBLK)
        def _():
          start_idx(blk + 2, p)

        @pl.when(blk + 1 < _NBLK)
        def _():
          wait_idx(1 - p)
          start_gather(1 - p)

        @pl.when(blk >= 2)
        def _():
          wait_out(p)

        compute(p, tr0, tr1)
        start_out(blk, p)

    return carry

  lax.fori_loop(0, (_NBLK + 1) // 2, it_body, 0)

  wait_out((_NBLK - 2) % 2)
  wait_out((_NBLK - 1) % 2)


@jax.jit
def _decode(z, src, dst):
  mesh = plsc.VectorSubcoreMesh(
      core_axis_name="c", subcore_axis_name="s",
      num_cores=_NUM_CORES, num_subcores=_NUM_SUBCORES)
  return pl.kernel(
      _sc_body,
      out_type=jax.ShapeDtypeStruct((_N_EDGES,), jnp.float32),
      mesh=mesh,
      compiler_params=pltpu.CompilerParams(needs_layout_passes=False),
      scratch_types=[
          pltpu.VMEM((_B,), jnp.int32),
          pltpu.VMEM((_B,), jnp.int32),
          pltpu.VMEM((_B,), jnp.int32),
          pltpu.VMEM((_B,), jnp.int32),
          pltpu.VMEM((_B, _D), jnp.float32),
          pltpu.VMEM((_B, _D), jnp.float32),
          pltpu.VMEM((_B, _D), jnp.float32),
          pltpu.VMEM((_B, _D), jnp.float32),
          pltpu.VMEM((_B,), jnp.float32),
          pltpu.VMEM((_B,), jnp.float32),
          pltpu.VMEM_SHARED((_N_NODES, _D), jnp.float32),
          pltpu.VMEM((_LANES, _LANES + 1), jnp.float32),
          pltpu.VMEM((_LANES, _LANES + 1), jnp.float32),
          pltpu.SemaphoreType.DMA,
          pltpu.SemaphoreType.DMA,
          pltpu.SemaphoreType.DMA,
          pltpu.SemaphoreType.DMA,
          pltpu.SemaphoreType.DMA,
          pltpu.SemaphoreType.DMA,
      ],
  )(z, src, dst)


def kernel(z, edge_index):
  src = edge_index[0]
  dst = edge_index[1]
  return _decode(z, src, dst)


# speedup vs baseline: 1.8128x; 1.2623x over previous
import jax
import jax.numpy as jnp
from jax import lax
from jax.experimental import pallas as pl
from jax.experimental.pallas import tpu as pltpu
from jax.experimental.pallas import tpu_sc as plsc

_NUM_CORES = 2
_NUM_SUBCORES = 16
_NUM_WORKERS = _NUM_CORES * _NUM_SUBCORES
_LANES = 16

_N_EDGES = 320000
_N_NODES = 10000
_D = 128
_B = 80
_EPW = _N_EDGES // _NUM_WORKERS
_NBLK = _EPW // _B
_ROWS_PER_TILE = 624


def _sc_body(z_hbm, src_hbm, dst_hbm, out_hbm,
             si0, si1, di0, di1, sr0, sr1, dr0, dr1, ob0, ob1, z_sh,
             tr0, tr1,
             sem_i0, sem_i1, sem_g0, sem_g1, sem_o0, sem_o1):
  sid = lax.axis_index("s")
  wid = lax.axis_index("c") * _NUM_SUBCORES + sid
  base = wid * _EPW

  si = (si0, si1)
  di = (di0, di1)
  sr = (sr0, sr1)
  dr = (dr0, dr1)
  ob = (ob0, ob1)
  sem_i = (sem_i0, sem_i1)
  sem_g = (sem_g0, sem_g1)
  sem_o = (sem_o0, sem_o1)

  r0 = sid * _ROWS_PER_TILE
  pltpu.sync_copy(z_hbm.at[pl.ds(r0, _ROWS_PER_TILE)],
                  z_sh.at[pl.ds(r0, _ROWS_PER_TILE)])
  rem0 = _NUM_SUBCORES * _ROWS_PER_TILE

  @pl.when(sid == 0)
  def _():
    pltpu.sync_copy(z_hbm.at[pl.ds(rem0, _N_NODES - rem0)],
                    z_sh.at[pl.ds(rem0, _N_NODES - rem0)])

  plsc.subcore_barrier()

  def start_idx(blk, p):
    off = base + blk * _B
    pltpu.async_copy(src_hbm.at[pl.ds(off, _B)], si[p], sem_i[p])
    pltpu.async_copy(dst_hbm.at[pl.ds(off, _B)], di[p], sem_i[p])

  def wait_idx(p):
    pltpu.make_async_copy(src_hbm.at[pl.ds(0, _B)], si[p], sem_i[p]).wait()
    pltpu.make_async_copy(dst_hbm.at[pl.ds(0, _B)], di[p], sem_i[p]).wait()

  def start_gather(p):
    pltpu.async_copy(z_sh.at[si[p]], sr[p], sem_g[p])
    pltpu.async_copy(z_sh.at[di[p]], dr[p], sem_g[p])

  def wait_gather(p):
    pltpu.make_async_copy(z_sh.at[si[p]], sr[p], sem_g[p]).wait()
    pltpu.make_async_copy(z_sh.at[di[p]], dr[p], sem_g[p]).wait()

  def start_out(blk, p):
    pltpu.async_copy(ob[p], out_hbm.at[pl.ds(base + blk * _B, _B)], sem_o[p])

  def wait_out(p):
    pltpu.make_async_copy(ob[p], out_hbm.at[pl.ds(0, _B)], sem_o[p]).wait()

  def compute(p, tr0, tr1):
    s_rows = sr[p]
    d_rows = dr[p]
    lane = lax.iota(jnp.int32, _LANES)

    def grp_body(g, carry2):
      e0 = g * _LANES

      @plsc.parallel_loop(e0, e0 + _LANES, unroll=4)
      def _(j):
        acc = (s_rows[j, pl.ds(0, _LANES)] * d_rows[j, pl.ds(0, _LANES)])
        for k in range(1, _D // _LANES):
          acc = acc + (s_rows[j, pl.ds(k * _LANES, _LANES)]
                       * d_rows[j, pl.ds(k * _LANES, _LANES)])
        tr0[j - e0, pl.ds(0, _LANES)] = acc
      u = [plsc.load_gather(tr0, [lane, jnp.full((_LANES,), k, jnp.int32)])
           for k in range(_LANES)]
      while len(u) > 1:
        u = [u[i2] + u[i2 + 1] for i2 in range(0, len(u), 2)]
      ob[p][pl.ds(e0, _LANES)] = 1.0 / (1.0 + jnp.exp(-u[0]))
      return carry2

    lax.fori_loop(0, _B // _LANES, grp_body, 0)

  start_idx(0, 0)
  start_idx(1, 1)
  wait_idx(0)
  start_gather(0)

  def it_body(i, carry):
    for p in range(2):
      blk = 2 * i + p

      @pl.when(blk < _NBLK)
      def _():
        wait_gather(p)

        @pl.when(blk + 2 < _NBLK)
        def _():
          start_idx(blk + 2, p)

        @pl.when(blk + 1 < _NBLK)
        def _():
          wait_idx(1 - p)
          start_gather(1 - p)

        @pl.when(blk >= 2)
        def _():
          wait_out(p)

        compute(p, tr0, tr1)
        start_out(blk, p)

    return carry

  lax.fori_loop(0, (_NBLK + 1) // 2, it_body, 0)

  wait_out((_NBLK - 2) % 2)
  wait_out((_NBLK - 1) % 2)


@jax.jit
def _decode(z, src, dst):
  mesh = plsc.VectorSubcoreMesh(
      core_axis_name="c", subcore_axis_name="s",
      num_cores=_NUM_CORES, num_subcores=_NUM_SUBCORES)
  return pl.kernel(
      _sc_body,
      out_type=jax.ShapeDtypeStruct((_N_EDGES,), jnp.float32),
      mesh=mesh,
      compiler_params=pltpu.CompilerParams(needs_layout_passes=False),
      scratch_types=[
          pltpu.VMEM((_B,), jnp.int32),
          pltpu.VMEM((_B,), jnp.int32),
          pltpu.VMEM((_B,), jnp.int32),
          pltpu.VMEM((_B,), jnp.int32),
          pltpu.VMEM((_B, _D), jnp.float32),
          pltpu.VMEM((_B, _D), jnp.float32),
          pltpu.VMEM((_B, _D), jnp.float32),
          pltpu.VMEM((_B, _D), jnp.float32),
          pltpu.VMEM((_B,), jnp.float32),
          pltpu.VMEM((_B,), jnp.float32),
          pltpu.VMEM_SHARED((_N_NODES, _D), jnp.float32),
          pltpu.VMEM((_LANES, _LANES + 1), jnp.float32),
          pltpu.VMEM((_LANES, _LANES + 1), jnp.float32),
          pltpu.SemaphoreType.DMA,
          pltpu.SemaphoreType.DMA,
          pltpu.SemaphoreType.DMA,
          pltpu.SemaphoreType.DMA,
          pltpu.SemaphoreType.DMA,
          pltpu.SemaphoreType.DMA,
      ],
  )(z, src, dst)


def kernel(z, edge_index):
  src = edge_index[0]
  dst = edge_index[1]
  return _decode(z, src, dst)
